# baseline (speedup 1.0000x reference)
import functools

import jax
import jax.numpy as jnp
from jax import lax
from jax.experimental import pallas as pl
from jax.experimental.pallas import tpu as pltpu
from jax.experimental.pallas import tpu_sc as plsc

N = 10000
E = 320000
LATENT = 128
STEPS = 15
TW = 5
OUT = 5
NTYPES = 9

NW = 32
N_PAD = 10240
E_PAD = 327680
B_PER_W = E_PAD // NW
CHUNK = 128
NCH = B_PER_W // CHUNK
ZROWS = N_PAD // 16

_mesh = lambda: plsc.VectorSubcoreMesh(core_axis_name="c", subcore_axis_name="s")


DEPTH = 2
GRP = 8
NGRP = NCH // GRP


def _make_gather2(D, ne=E_PAD, off=0):
    nch = (ne // NW) // CHUNK
    ngrp = nch // GRP
    off_row = off // CHUNK

    def body(table, sidx, ridx, gs_out, gr_out, idxs, *rest):
        rows = rest[:DEPTH]
        gsems = rest[DEPTH:2 * DEPTH]
        ssems = rest[2 * DEPTH:3 * DEPTH]
        stab = rest[3 * DEPTH]
        cid = lax.axis_index("c")
        sid = lax.axis_index("s")
        base_row = (cid * 16 + sid) * nch
        pltpu.sync_copy(table.at[pl.ds(sid * (N_PAD // 16), N_PAD // 16)],
                        stab.at[pl.ds(sid * (N_PAD // 16), N_PAD // 16)])
        plsc.subcore_barrier()

        def do(idx2, out_hbm):
            def group(g, carry):
                row0 = base_row + g * GRP
                pltpu.sync_copy(idx2.at[pl.ds(off_row + row0, GRP)], idxs)
                for w in range(GRP // DEPTH):
                    handles = []
                    for b in range(DEPTH):
                        if w == 0:
                            @pl.when(g > 0)
                            def _wait_store(b=b):
                                pltpu.make_async_copy(
                                    rows[b], out_hbm.at[pl.ds(0, CHUNK)],
                                    ssems[b]).wait()
                        else:
                            pltpu.make_async_copy(
                                rows[b], out_hbm.at[pl.ds(0, CHUNK)],
                                ssems[b]).wait()
                        handles.append(pltpu.async_copy(
                            stab.at[idxs.at[w * DEPTH + b, 0]], rows[b],
                            gsems[b]))
                    for b in range(DEPTH):
                        handles[b].wait()
                        pltpu.async_copy(
                            rows[b],
                            out_hbm.at[pl.ds((row0 + w * DEPTH + b) * CHUNK,
                                             CHUNK)], ssems[b])
                return carry

            lax.fori_loop(0, ngrp, group, 0)
            for b in range(DEPTH):
                pltpu.make_async_copy(
                    rows[b], out_hbm.at[pl.ds(0, CHUNK)], ssems[b]).wait()

        do(sidx, gs_out)
        do(ridx, gr_out)

    return functools.partial(
        pl.kernel,
        out_type=(jax.ShapeDtypeStruct((ne, D), jnp.float32),
                  jax.ShapeDtypeStruct((ne, D), jnp.float32)),
        mesh=_mesh(),
        scratch_types=[pltpu.VMEM((GRP, 1, CHUNK), jnp.int32)]
        + [pltpu.VMEM((CHUNK, D), jnp.float32) for _ in range(DEPTH)]
        + [pltpu.SemaphoreType.DMA for _ in range(2 * DEPTH)]
        + [pltpu.VMEM_SHARED((N_PAD, D), jnp.float32)],
    )(body)


SDEPTH = 2


def _make_scatter(ne=E_PAD, off=0):
    nch = (ne // NW) // CHUNK
    ngrp = nch // GRP
    off_row = off // CHUNK

    def body(new_el, ridx, zrows, out, idxs, *rest):
        rows = rest[:SDEPTH]
        lsems = rest[SDEPTH:2 * SDEPTH]
        shared = rest[2 * SDEPTH]
        cid = lax.axis_index("c")
        sid = lax.axis_index("s")
        base_row = (cid * 16 + sid) * nch
        pltpu.sync_copy(zrows, shared.at[pl.ds(sid * ZROWS, ZROWS)])
        plsc.subcore_barrier()

        def group(g, carry):
            row0 = base_row + g * GRP
            pltpu.sync_copy(ridx.at[pl.ds(off_row + row0, GRP)], idxs)
            for w in range(GRP // SDEPTH):
                handles = [
                    pltpu.async_copy(
                        new_el.at[pl.ds((row0 + w * SDEPTH + b) * CHUNK,
                                        CHUNK)], rows[b], lsems[b])
                    for b in range(SDEPTH)
                ]
                for b in range(SDEPTH):
                    handles[b].wait()
                    pltpu.sync_copy(rows[b],
                                    shared.at[idxs.at[w * SDEPTH + b, 0]],
                                    add=True)
            return carry

        lax.fori_loop(0, ngrp, group, 0)
        plsc.subcore_barrier()
        pltpu.sync_copy(shared.at[pl.ds(sid * ZROWS, ZROWS)],
                        out.at[cid, pl.ds(sid * ZROWS, ZROWS)])

    return functools.partial(
        pl.kernel,
        out_type=jax.ShapeDtypeStruct((2, N_PAD, LATENT), jnp.float32),
        mesh=_mesh(),
        scratch_types=[pltpu.VMEM((GRP, 1, CHUNK), jnp.int32)]
        + [pltpu.VMEM((CHUNK, LATENT), jnp.float32) for _ in range(SDEPTH)]
        + [pltpu.SemaphoreType.DMA for _ in range(SDEPTH)]
        + [pltpu.VMEM_SHARED((N_PAD, LATENT), jnp.float32)],
    )(body)


def _ln(h, g, bt):
    mu = jnp.mean(h, axis=-1, keepdims=True)
    hm = h - mu
    var = jnp.mean(hm * hm, axis=-1, keepdims=True)
    return hm * jax.lax.rsqrt(var + 1e-5) * g + bt


def _mm(a, b):
    return jnp.dot(a, b, preferred_element_type=jnp.float32)


def _full(shape):
    return pl.BlockSpec(shape, lambda i: (0, 0))


def _node_enc_body(base_ref, nt_ref, w1, b1, w2, b2, g, bt, out_ref):
    nt = nt_ref[...]
    io = lax.broadcasted_iota(jnp.int32, base_ref.shape, 1)
    oh = jnp.where((io >= 5) & (io - 5 == nt), 1.0, 0.0)
    x = base_ref[...] + oh
    h = jnp.maximum(_mm(x, w1[...]) + b1[...], 0.0)
    h = jnp.maximum(_mm(h, w2[...]) + b2[...], 0.0)
    out_ref[...] = _ln(h, g[...], bt[...])


def _edge_enc_body(gs_ref, gr_ref, w1d, wdm, wdw, b1, w2, b2, g, bt, out_ref):
    d = gs_ref[...] - gr_ref[...]
    m0 = d[:, 0:1]
    m1 = d[:, 1:2]
    w0 = d[:, 2:3]
    w1_ = d[:, 3:4]
    dm = jnp.sqrt(m0 * m0 + m1 * m1)
    dw = jnp.sqrt(w0 * w0 + w1_ * w1_)
    h = _mm(d, w1d[...]) + dm * wdm[...] + dw * wdw[...] + b1[...]
    h = jnp.maximum(h, 0.0)
    h = jnp.maximum(_mm(h, w2[...]) + b2[...], 0.0)
    out_ref[...] = _ln(h, g[...], bt[...])


def _edge_step_body(ns_ref, nr_ref, el_ref, w1a, w1b, w1c, b1, w2, b2, g, bt,
                    elo_ref):
    el = el_ref[...]
    h = (_mm(ns_ref[...], w1a[...]) + _mm(nr_ref[...], w1b[...])
         + _mm(el, w1c[...]) + b1[...])
    h = jnp.maximum(h, 0.0)
    h = jnp.maximum(_mm(h, w2[...]) + b2[...], 0.0)
    elo_ref[...] = el + _ln(h, g[...], bt[...])


def _sum4_body(p0_ref, p1_ref, p2_ref, p3_ref, out_ref):
    out_ref[...] = (p0_ref[...] + p1_ref[...]) + (p2_ref[...] + p3_ref[...])


def _node_step_body(nl_ref, p0_ref, p1_ref, p2_ref, p3_ref, tprev_ref, w1a,
                    w1b, b1, w2, b2, g, bt, out_ref, tsum_ref):
    nl = nl_ref[...]
    tsum = (p0_ref[...] + p1_ref[...]) + (p2_ref[...] + p3_ref[...])
    tsum_ref[...] = tsum
    aggr = tsum - tprev_ref[...]
    h = _mm(nl, w1a[...]) + _mm(aggr, w1b[...]) + b1[...]
    h = jnp.maximum(h, 0.0)
    h = jnp.maximum(_mm(h, w2[...]) + b2[...], 0.0)
    out_ref[...] = nl + _ln(h, g[...], bt[...])


def _decoder_body(nl_ref, w1, b1, w2, b2, out_ref):
    h = _mm(nl_ref[...], w1[...]) + b1[...]
    h = h * jax.nn.sigmoid(h)
    out_ref[...] = _mm(h, w2[...]) + b2[...]


def _row(v):
    return v.reshape(1, -1)


BE = 2048
NBE = E_PAD // BE
BN = 1024
NBN = N_PAD // BN


def _edge_spec():
    return pl.BlockSpec((BE, LATENT), lambda i: (i, 0))


def _node_spec(w=LATENT):
    return pl.BlockSpec((BN, w), lambda i: (i, 0))


def kernel(pvf, mat_param_D, mat_param_X, mesh_pos, world_pos, node_type,
           senders, receivers, params):
    f32 = jnp.float32
    s = jnp.concatenate([senders.astype(jnp.int32),
                         jnp.zeros((E_PAD - E,), jnp.int32)])
    r = jnp.concatenate([receivers.astype(jnp.int32),
                         jnp.full((E_PAD - E,), N, jnp.int32)])
    s2 = s.reshape(E_PAD // CHUNK, 1, CHUNK)
    r2 = r.reshape(E_PAD // CHUNK, 1, CHUNK)

    geo = jnp.concatenate(
        [mesh_pos, world_pos, pvf, jnp.zeros((N, 121), f32)], axis=1)
    geo = jnp.concatenate([geo, jnp.zeros((N_PAD - N, 128), f32)], axis=0)

    base = jnp.concatenate(
        [pvf, mat_param_D, mat_param_X, jnp.zeros((N, 11), f32)], axis=1)
    base = jnp.concatenate([base, jnp.zeros((N_PAD - N, 16), f32)], axis=0)
    nt = jnp.concatenate([node_type.astype(jnp.int32),
                          jnp.zeros((N_PAD - N,), jnp.int32)]).reshape(-1, 1)

    ne_w1, ne_b1, ne_w2, ne_b2, ne_g, ne_bt = params["node_enc"]
    ne_w1p = jnp.concatenate([ne_w1, jnp.zeros((2, LATENT), f32)], axis=0)

    ee_w1, ee_b1, ee_w2, ee_b2, ee_g, ee_bt = params["edge_enc"]
    w1d = jnp.concatenate(
        [ee_w1[0:2], ee_w1[3:5], ee_w1[6:9], jnp.zeros((121, LATENT), f32)],
        axis=0)
    wdm = ee_w1[2:3]
    wdw = ee_w1[5:6]

    be_w1, be_b1, be_w2, be_b2, be_g, be_bt = params["blk_edge"]
    bn_w1, bn_b1, bn_w2, bn_b2, bn_g, bn_bt = params["blk_node"]

    dt = jnp.repeat(jnp.arange(1, TW + 1), OUT).astype(f32)
    dw2 = params["dec_W2"] * dt[None, :]
    db2 = params["dec_b2"] * dt
    dw2 = jnp.concatenate([dw2, jnp.zeros((8, 128 - OUT * TW), f32)], axis=1)
    db2 = jnp.concatenate([db2, jnp.zeros((128 - OUT * TW,), f32)])

    EH = E_PAD // 2
    gather_geo = _make_gather2(LATENT)
    gather_nl = gather_geo
    scatter_h0 = _make_scatter(EH, 0)
    scatter_h1 = _make_scatter(EH, EH)
    zrows = jnp.zeros((ZROWS, LATENT), f32)
    NBH = EH // BE

    def _edge_spec_h():
        return pl.BlockSpec((BE, LATENT), lambda i: (i, 0))

    nl = pl.pallas_call(
        _node_enc_body,
        grid=(NBN,),
        in_specs=[_node_spec(16), pl.BlockSpec((BN, 1), lambda i: (i, 0)),
                  _full((16, LATENT)), _full((1, LATENT)),
                  _full((LATENT, LATENT)), _full((1, LATENT)),
                  _full((1, LATENT)), _full((1, LATENT))],
        out_specs=_node_spec(),
        out_shape=jax.ShapeDtypeStruct((N_PAD, LATENT), f32),
    )(base, nt, ne_w1p, _row(ne_b1), ne_w2, _row(ne_b2), _row(ne_g),
      _row(ne_bt))

    gs, gr = gather_geo(geo, s2, r2)
    el = pl.pallas_call(
        _edge_enc_body,
        grid=(NBE,),
        in_specs=[_edge_spec(), _edge_spec(),
                  _full((LATENT, LATENT)), _full((1, LATENT)), _full((1, LATENT)),
                  _full((1, LATENT)), _full((LATENT, LATENT)),
                  _full((1, LATENT)), _full((1, LATENT)), _full((1, LATENT))],
        out_specs=_edge_spec(),
        out_shape=jax.ShapeDtypeStruct((E_PAD, LATENT), f32),
    )(gs, gr, w1d, wdm, wdw, _row(ee_b1), ee_w2, _row(ee_b2), _row(ee_g),
      _row(ee_bt))

    def _edge_spec_off(o):
        return pl.BlockSpec((BE, LATENT), lambda i: (i + o, 0))

    edge_steps = [
        pl.pallas_call(
            _edge_step_body,
            grid=(NBH,),
            in_specs=[_edge_spec_off(h * NBH), _edge_spec_off(h * NBH),
                      _edge_spec_h(),
                      _full((LATENT, LATENT)), _full((LATENT, LATENT)),
                      _full((LATENT, LATENT)), _full((1, LATENT)),
                      _full((LATENT, LATENT)), _full((1, LATENT)),
                      _full((1, LATENT)), _full((1, LATENT))],
            out_specs=_edge_spec_h(),
            out_shape=jax.ShapeDtypeStruct((EH, LATENT), f32),
            input_output_aliases={2: 0},
        )
        for h in (0, 1)
    ]

    node_step = pl.pallas_call(
        _node_step_body,
        grid=(NBN,),
        in_specs=[_node_spec(), _node_spec(), _node_spec(), _node_spec(),
                  _node_spec(), _node_spec(),
                  _full((LATENT, LATENT)), _full((LATENT, LATENT)),
                  _full((1, LATENT)), _full((LATENT, LATENT)),
                  _full((1, LATENT)), _full((1, LATENT)), _full((1, LATENT))],
        out_specs=(_node_spec(), _node_spec()),
        out_shape=(jax.ShapeDtypeStruct((N_PAD, LATENT), f32),
                   jax.ShapeDtypeStruct((N_PAD, LATENT), f32)),
        input_output_aliases={0: 0},
    )

    sum4 = pl.pallas_call(
        _sum4_body,
        grid=(NBN,),
        in_specs=[_node_spec(), _node_spec(), _node_spec(), _node_spec()],
        out_specs=_node_spec(),
        out_shape=jax.ShapeDtypeStruct((N_PAD, LATENT), f32),
    )

    def step(carry, w):
        nl, el0, el1, tprev = carry
        (ew1, eb1, ew2, eb2, eg, ebt, nw1, nb1, nw2, nb2, ng, nbt) = w
        ew = (ew1[:LATENT], ew1[LATENT:2 * LATENT], ew1[2 * LATENT:],
              _row(eb1), ew2, _row(eb2), _row(eg), _row(ebt))
        ns, nr = gather_nl(nl, s2, r2)
        el0 = edge_steps[0](ns, nr, el0, *ew)
        el1 = edge_steps[1](ns, nr, el1, *ew)
        parts0 = scatter_h0(el0, r2, zrows)
        parts1 = scatter_h1(el1, r2, zrows)
        nl, tsum = node_step(nl, parts0[0], parts0[1], parts1[0], parts1[1],
                             tprev, nw1[:LATENT], nw1[LATENT:], _row(nb1),
                             nw2, _row(nb2), _row(ng), _row(nbt))
        return (nl, el0, el1, tsum), None

    el0 = el[:EH]
    el1 = el[EH:]
    parts0 = scatter_h0(el0, r2, zrows)
    parts1 = scatter_h1(el1, r2, zrows)
    t0 = sum4(parts0[0], parts0[1], parts1[0], parts1[1])
    (nl, el0, el1, t0), _ = lax.scan(
        step, (nl, el0, el1, t0),
        (be_w1, be_b1, be_w2, be_b2, be_g, be_bt,
         bn_w1, bn_b1, bn_w2, bn_b2, bn_g, bn_bt))

    dec = pl.pallas_call(
        _decoder_body,
        grid=(NBN,),
        in_specs=[_node_spec(), _full((LATENT, 8)), _full((1, 8)),
                  _full((8, 128)), _full((1, 128))],
        out_specs=_node_spec(128),
        out_shape=jax.ShapeDtypeStruct((N_PAD, 128), f32),
    )(nl, params["dec_W1"], _row(params["dec_b1"]), dw2, _row(db2))

    decoded = dec[:N, :OUT * TW]
    return decoded.reshape(N, TW, OUT).transpose(1, 0, 2)


# speedup vs baseline: 1.1201x; 1.1201x over previous
import functools

import jax
import jax.numpy as jnp
from jax import lax
from jax.experimental import pallas as pl
from jax.experimental.pallas import tpu as pltpu
from jax.experimental.pallas import tpu_sc as plsc

N = 10000
E = 320000
LATENT = 128
STEPS = 15
TW = 5
OUT = 5
NTYPES = 9

NW = 32
N_PAD = 10240
E_PAD = 327680
B_PER_W = E_PAD // NW
CHUNK = 128
NCH = B_PER_W // CHUNK
ZROWS = N_PAD // 16

_mesh = lambda: plsc.VectorSubcoreMesh(core_axis_name="c", subcore_axis_name="s")


DEPTH = 2
GRP = 8
NGRP = NCH // GRP


def _make_gather2(D, ne=E_PAD, off=0, ab=False):
    per_core = 16 if ab else NW
    nch = (ne // per_core) // CHUNK
    ngrp = nch // GRP
    off_row = off // CHUNK

    def body(table, tableb, sidx, ridx, gs_out, gr_out, idxs, *rest):
        rows = rest[:DEPTH]
        gsems = rest[DEPTH:2 * DEPTH]
        ssems = rest[2 * DEPTH:3 * DEPTH]
        stab = rest[3 * DEPTH]
        cid = lax.axis_index("c")
        sid = lax.axis_index("s")
        if ab:
            base_row = sid * nch
        else:
            base_row = (cid * 16 + sid) * nch
        if ab:
            @pl.when(cid == 0)
            def _stage_a():
                pltpu.sync_copy(
                    table.at[pl.ds(sid * (N_PAD // 16), N_PAD // 16)],
                    stab.at[pl.ds(sid * (N_PAD // 16), N_PAD // 16)])

            @pl.when(cid == 1)
            def _stage_b():
                pltpu.sync_copy(
                    tableb.at[pl.ds(sid * (N_PAD // 16), N_PAD // 16)],
                    stab.at[pl.ds(sid * (N_PAD // 16), N_PAD // 16)])
        else:
            pltpu.sync_copy(table.at[pl.ds(sid * (N_PAD // 16), N_PAD // 16)],
                            stab.at[pl.ds(sid * (N_PAD // 16), N_PAD // 16)])
        plsc.subcore_barrier()

        def do(idx2, out_hbm):
            def group(g, carry):
                row0 = base_row + g * GRP
                pltpu.sync_copy(idx2.at[pl.ds(off_row + row0, GRP)], idxs)
                for w in range(GRP // DEPTH):
                    handles = []
                    for b in range(DEPTH):
                        if w == 0:
                            @pl.when(g > 0)
                            def _wait_store(b=b):
                                pltpu.make_async_copy(
                                    rows[b], out_hbm.at[pl.ds(0, CHUNK)],
                                    ssems[b]).wait()
                        else:
                            pltpu.make_async_copy(
                                rows[b], out_hbm.at[pl.ds(0, CHUNK)],
                                ssems[b]).wait()
                        handles.append(pltpu.async_copy(
                            stab.at[idxs.at[w * DEPTH + b, 0]], rows[b],
                            gsems[b]))
                    for b in range(DEPTH):
                        handles[b].wait()
                        pltpu.async_copy(
                            rows[b],
                            out_hbm.at[pl.ds((row0 + w * DEPTH + b) * CHUNK,
                                             CHUNK)], ssems[b])
                return carry

            lax.fori_loop(0, ngrp, group, 0)
            for b in range(DEPTH):
                pltpu.make_async_copy(
                    rows[b], out_hbm.at[pl.ds(0, CHUNK)], ssems[b]).wait()

        if ab:
            @pl.when(cid == 0)
            def _do_s():
                do(sidx, gs_out)

            @pl.when(cid == 1)
            def _do_r():
                do(ridx, gr_out)
        else:
            do(sidx, gs_out)
            do(ridx, gr_out)

    return functools.partial(
        pl.kernel,
        out_type=(jax.ShapeDtypeStruct((ne, D), jnp.float32),
                  jax.ShapeDtypeStruct((ne, D), jnp.float32)),
        mesh=_mesh(),
        scratch_types=[pltpu.VMEM((GRP, 1, CHUNK), jnp.int32)]
        + [pltpu.VMEM((CHUNK, D), jnp.float32) for _ in range(DEPTH)]
        + [pltpu.SemaphoreType.DMA for _ in range(2 * DEPTH)]
        + [pltpu.VMEM_SHARED((N_PAD, D), jnp.float32)],
    )(body)


SDEPTH = 2


def _make_scatter(ne=E_PAD, off=0):
    nch = (ne // NW) // CHUNK
    ngrp = nch // GRP
    off_row = off // CHUNK

    def body(new_el, ridx, zrows, out, idxs, *rest):
        rows = rest[:SDEPTH]
        lsems = rest[SDEPTH:2 * SDEPTH]
        shared = rest[2 * SDEPTH]
        cid = lax.axis_index("c")
        sid = lax.axis_index("s")
        base_row = (cid * 16 + sid) * nch
        pltpu.sync_copy(zrows, shared.at[pl.ds(sid * ZROWS, ZROWS)])
        plsc.subcore_barrier()

        def group(g, carry):
            row0 = base_row + g * GRP
            pltpu.sync_copy(ridx.at[pl.ds(off_row + row0, GRP)], idxs)
            for w in range(GRP // SDEPTH):
                handles = [
                    pltpu.async_copy(
                        new_el.at[pl.ds((row0 + w * SDEPTH + b) * CHUNK,
                                        CHUNK)], rows[b], lsems[b])
                    for b in range(SDEPTH)
                ]
                for b in range(SDEPTH):
                    handles[b].wait()
                    pltpu.sync_copy(rows[b],
                                    shared.at[idxs.at[w * SDEPTH + b, 0]],
                                    add=True)
            return carry

        lax.fori_loop(0, ngrp, group, 0)
        plsc.subcore_barrier()
        pltpu.sync_copy(shared.at[pl.ds(sid * ZROWS, ZROWS)],
                        out.at[cid, pl.ds(sid * ZROWS, ZROWS)])

    return functools.partial(
        pl.kernel,
        out_type=jax.ShapeDtypeStruct((2, N_PAD, LATENT), jnp.float32),
        mesh=_mesh(),
        scratch_types=[pltpu.VMEM((GRP, 1, CHUNK), jnp.int32)]
        + [pltpu.VMEM((CHUNK, LATENT), jnp.float32) for _ in range(SDEPTH)]
        + [pltpu.SemaphoreType.DMA for _ in range(SDEPTH)]
        + [pltpu.VMEM_SHARED((N_PAD, LATENT), jnp.float32)],
    )(body)


def _ln(h, g, bt):
    mu = jnp.mean(h, axis=-1, keepdims=True)
    hm = h - mu
    var = jnp.mean(hm * hm, axis=-1, keepdims=True)
    return hm * jax.lax.rsqrt(var + 1e-5) * g + bt


def _mm(a, b):
    return jnp.dot(a, b, preferred_element_type=jnp.float32)


def _full(shape):
    return pl.BlockSpec(shape, lambda i: (0, 0))


def _node_enc_body(base_ref, nt_ref, w1, b1, w2, b2, g, bt, out_ref):
    nt = nt_ref[...]
    io = lax.broadcasted_iota(jnp.int32, base_ref.shape, 1)
    oh = jnp.where((io >= 5) & (io - 5 == nt), 1.0, 0.0)
    x = base_ref[...] + oh
    h = jnp.maximum(_mm(x, w1[...]) + b1[...], 0.0)
    h = jnp.maximum(_mm(h, w2[...]) + b2[...], 0.0)
    out_ref[...] = _ln(h, g[...], bt[...])


def _edge_enc_body(gs_ref, gr_ref, w1d, wdm, wdw, b1, w2, b2, g, bt, out_ref):
    d = gs_ref[...] - gr_ref[...]
    m0 = d[:, 0:1]
    m1 = d[:, 1:2]
    w0 = d[:, 2:3]
    w1_ = d[:, 3:4]
    dm = jnp.sqrt(m0 * m0 + m1 * m1)
    dw = jnp.sqrt(w0 * w0 + w1_ * w1_)
    h = _mm(d, w1d[...]) + dm * wdm[...] + dw * wdw[...] + b1[...]
    h = jnp.maximum(h, 0.0)
    h = jnp.maximum(_mm(h, w2[...]) + b2[...], 0.0)
    out_ref[...] = _ln(h, g[...], bt[...])


def _edge_step_body(ns_ref, nr_ref, el_ref, w1c, b1, w2, b2, g, bt,
                    elo_ref):
    el = el_ref[...]
    h = (ns_ref[...] + nr_ref[...]) + (_mm(el, w1c[...]) + b1[...])
    h = jnp.maximum(h, 0.0)
    h = jnp.maximum(_mm(h, w2[...]) + b2[...], 0.0)
    elo_ref[...] = el + _ln(h, g[...], bt[...])


def _ab_body(nl_ref, w1a, w1b, a_ref, b_ref):
    nl = nl_ref[...]
    a_ref[...] = _mm(nl, w1a[...])
    b_ref[...] = _mm(nl, w1b[...])


def _sum4_body(p0_ref, p1_ref, p2_ref, p3_ref, out_ref):
    out_ref[...] = (p0_ref[...] + p1_ref[...]) + (p2_ref[...] + p3_ref[...])


def _node_step_body(nl_ref, p0_ref, p1_ref, p2_ref, p3_ref, tprev_ref, w1a,
                    w1b, b1, w2, b2, g, bt, out_ref, tsum_ref):
    nl = nl_ref[...]
    tsum = (p0_ref[...] + p1_ref[...]) + (p2_ref[...] + p3_ref[...])
    tsum_ref[...] = tsum
    aggr = tsum - tprev_ref[...]
    h = _mm(nl, w1a[...]) + _mm(aggr, w1b[...]) + b1[...]
    h = jnp.maximum(h, 0.0)
    h = jnp.maximum(_mm(h, w2[...]) + b2[...], 0.0)
    out_ref[...] = nl + _ln(h, g[...], bt[...])


def _decoder_body(nl_ref, w1, b1, w2, b2, out_ref):
    h = _mm(nl_ref[...], w1[...]) + b1[...]
    h = h * jax.nn.sigmoid(h)
    out_ref[...] = _mm(h, w2[...]) + b2[...]


def _row(v):
    return v.reshape(1, -1)


BE = 2048
NBE = E_PAD // BE
BN = 1024
NBN = N_PAD // BN


def _edge_spec():
    return pl.BlockSpec((BE, LATENT), lambda i: (i, 0))


def _node_spec(w=LATENT):
    return pl.BlockSpec((BN, w), lambda i: (i, 0))


def kernel(pvf, mat_param_D, mat_param_X, mesh_pos, world_pos, node_type,
           senders, receivers, params):
    f32 = jnp.float32
    s = jnp.concatenate([senders.astype(jnp.int32),
                         jnp.zeros((E_PAD - E,), jnp.int32)])
    r = jnp.concatenate([receivers.astype(jnp.int32),
                         jnp.full((E_PAD - E,), N, jnp.int32)])
    s2 = s.reshape(E_PAD // CHUNK, 1, CHUNK)
    r2 = r.reshape(E_PAD // CHUNK, 1, CHUNK)

    geo = jnp.concatenate(
        [mesh_pos, world_pos, pvf, jnp.zeros((N, 121), f32)], axis=1)
    geo = jnp.concatenate([geo, jnp.zeros((N_PAD - N, 128), f32)], axis=0)

    base = jnp.concatenate(
        [pvf, mat_param_D, mat_param_X, jnp.zeros((N, 11), f32)], axis=1)
    base = jnp.concatenate([base, jnp.zeros((N_PAD - N, 16), f32)], axis=0)
    nt = jnp.concatenate([node_type.astype(jnp.int32),
                          jnp.zeros((N_PAD - N,), jnp.int32)]).reshape(-1, 1)

    ne_w1, ne_b1, ne_w2, ne_b2, ne_g, ne_bt = params["node_enc"]
    ne_w1p = jnp.concatenate([ne_w1, jnp.zeros((2, LATENT), f32)], axis=0)

    ee_w1, ee_b1, ee_w2, ee_b2, ee_g, ee_bt = params["edge_enc"]
    w1d = jnp.concatenate(
        [ee_w1[0:2], ee_w1[3:5], ee_w1[6:9], jnp.zeros((121, LATENT), f32)],
        axis=0)
    wdm = ee_w1[2:3]
    wdw = ee_w1[5:6]

    be_w1, be_b1, be_w2, be_b2, be_g, be_bt = params["blk_edge"]
    bn_w1, bn_b1, bn_w2, bn_b2, bn_g, bn_bt = params["blk_node"]

    dt = jnp.repeat(jnp.arange(1, TW + 1), OUT).astype(f32)
    dw2 = params["dec_W2"] * dt[None, :]
    db2 = params["dec_b2"] * dt
    dw2 = jnp.concatenate([dw2, jnp.zeros((8, 128 - OUT * TW), f32)], axis=1)
    db2 = jnp.concatenate([db2, jnp.zeros((128 - OUT * TW,), f32)])

    EH = E_PAD // 2
    gather_geo = _make_gather2(LATENT)
    gather_h0 = _make_gather2(LATENT, EH, 0, ab=True)
    gather_h1 = _make_gather2(LATENT, EH, EH, ab=True)
    scatter_h0 = _make_scatter(EH, 0)
    scatter_h1 = _make_scatter(EH, EH)
    zrows = jnp.zeros((ZROWS, LATENT), f32)
    NBH = EH // BE

    def _edge_spec_h():
        return pl.BlockSpec((BE, LATENT), lambda i: (i, 0))

    nl = pl.pallas_call(
        _node_enc_body,
        grid=(NBN,),
        in_specs=[_node_spec(16), pl.BlockSpec((BN, 1), lambda i: (i, 0)),
                  _full((16, LATENT)), _full((1, LATENT)),
                  _full((LATENT, LATENT)), _full((1, LATENT)),
                  _full((1, LATENT)), _full((1, LATENT))],
        out_specs=_node_spec(),
        out_shape=jax.ShapeDtypeStruct((N_PAD, LATENT), f32),
    )(base, nt, ne_w1p, _row(ne_b1), ne_w2, _row(ne_b2), _row(ne_g),
      _row(ne_bt))

    gs, gr = gather_geo(geo, geo, s2, r2)
    el = pl.pallas_call(
        _edge_enc_body,
        grid=(NBE,),
        in_specs=[_edge_spec(), _edge_spec(),
                  _full((LATENT, LATENT)), _full((1, LATENT)), _full((1, LATENT)),
                  _full((1, LATENT)), _full((LATENT, LATENT)),
                  _full((1, LATENT)), _full((1, LATENT)), _full((1, LATENT))],
        out_specs=_edge_spec(),
        out_shape=jax.ShapeDtypeStruct((E_PAD, LATENT), f32),
    )(gs, gr, w1d, wdm, wdw, _row(ee_b1), ee_w2, _row(ee_b2), _row(ee_g),
      _row(ee_bt))

    edge_step = pl.pallas_call(
        _edge_step_body,
        grid=(NBH,),
        in_specs=[_edge_spec_h(), _edge_spec_h(), _edge_spec_h(),
                  _full((LATENT, LATENT)), _full((1, LATENT)),
                  _full((LATENT, LATENT)), _full((1, LATENT)),
                  _full((1, LATENT)), _full((1, LATENT))],
        out_specs=_edge_spec_h(),
        out_shape=jax.ShapeDtypeStruct((EH, LATENT), f32),
        input_output_aliases={2: 0},
    )

    ab_step = pl.pallas_call(
        _ab_body,
        grid=(NBN,),
        in_specs=[_node_spec(), _full((LATENT, LATENT)),
                  _full((LATENT, LATENT))],
        out_specs=(_node_spec(), _node_spec()),
        out_shape=(jax.ShapeDtypeStruct((N_PAD, LATENT), f32),
                   jax.ShapeDtypeStruct((N_PAD, LATENT), f32)),
    )

    node_step = pl.pallas_call(
        _node_step_body,
        grid=(NBN,),
        in_specs=[_node_spec(), _node_spec(), _node_spec(), _node_spec(),
                  _node_spec(), _node_spec(),
                  _full((LATENT, LATENT)), _full((LATENT, LATENT)),
                  _full((1, LATENT)), _full((LATENT, LATENT)),
                  _full((1, LATENT)), _full((1, LATENT)), _full((1, LATENT))],
        out_specs=(_node_spec(), _node_spec()),
        out_shape=(jax.ShapeDtypeStruct((N_PAD, LATENT), f32),
                   jax.ShapeDtypeStruct((N_PAD, LATENT), f32)),
        input_output_aliases={0: 0},
    )

    sum4 = pl.pallas_call(
        _sum4_body,
        grid=(NBN,),
        in_specs=[_node_spec(), _node_spec(), _node_spec(), _node_spec()],
        out_specs=_node_spec(),
        out_shape=jax.ShapeDtypeStruct((N_PAD, LATENT), f32),
    )

    def step(carry, w):
        nl, el0, el1, tprev = carry
        (ew1, eb1, ew2, eb2, eg, ebt, nw1, nb1, nw2, nb2, ng, nbt) = w
        ew = (ew1[2 * LATENT:],
              _row(eb1), ew2, _row(eb2), _row(eg), _row(ebt))
        A, B = ab_step(nl, ew1[:LATENT], ew1[LATENT:2 * LATENT])
        ns0, nr0 = gather_h0(A, B, s2, r2)
        ns1, nr1 = gather_h1(A, B, s2, r2)
        el0 = edge_step(ns0, nr0, el0, *ew)
        el1 = edge_step(ns1, nr1, el1, *ew)
        parts0 = scatter_h0(el0, r2, zrows)
        parts1 = scatter_h1(el1, r2, zrows)
        nl, tsum = node_step(nl, parts0[0], parts0[1], parts1[0], parts1[1],
                             tprev, nw1[:LATENT], nw1[LATENT:], _row(nb1),
                             nw2, _row(nb2), _row(ng), _row(nbt))
        return (nl, el0, el1, tsum), None

    el0 = el[:EH]
    el1 = el[EH:]
    parts0 = scatter_h0(el0, r2, zrows)
    parts1 = scatter_h1(el1, r2, zrows)
    t0 = sum4(parts0[0], parts0[1], parts1[0], parts1[1])
    (nl, el0, el1, t0), _ = lax.scan(
        step, (nl, el0, el1, t0),
        (be_w1, be_b1, be_w2, be_b2, be_g, be_bt,
         bn_w1, bn_b1, bn_w2, bn_b2, bn_g, bn_bt))

    dec = pl.pallas_call(
        _decoder_body,
        grid=(NBN,),
        in_specs=[_node_spec(), _full((LATENT, 8)), _full((1, 8)),
                  _full((8, 128)), _full((1, 128))],
        out_specs=_node_spec(128),
        out_shape=jax.ShapeDtypeStruct((N_PAD, 128), f32),
    )(nl, params["dec_W1"], _row(params["dec_b1"]), dw2, _row(db2))

    decoded = dec[:N, :OUT * TW]
    return decoded.reshape(N, TW, OUT).transpose(1, 0, 2)
